# Initial kernel scaffold; baseline (speedup 1.0000x reference)
import functools

import jax
import jax.numpy as jnp
from jax import lax
from jax.experimental import pallas as pl
from jax.experimental.pallas import tpu as pltpu
from jax.experimental.pallas import tpu_sc as plsc

V = 1000000
E = 16
B = 16384
S = 200

NC = 2
NS = 16
NW = NC * NS
ROWS_W = B // NW
CH = 128
NCHUNK = ROWS_W // CH

STAGE = 62496
STAGE_TAIL_OFF = NS * STAGE
STAGE_TAIL = V - STAGE_TAIL_OFF


ROWS128 = V * E // 128
TC_BLK = 5000


def _tc_body(tab_ref, m2_ref, bias_ref, out_ref):
    out_ref[...] = (
        lax.dot_general(tab_ref[...], m2_ref[...], (((1,), (0,)), ((), ())),
                        preferred_element_type=jnp.float32)
        + bias_ref[0, 0]
    )


def _fold_table(table128, m2, bias):
    return pl.pallas_call(
        _tc_body,
        grid=(ROWS128 // TC_BLK,),
        in_specs=[
            pl.BlockSpec((TC_BLK, 128), lambda i: (i, 0)),
            pl.BlockSpec((128, 8), lambda i: (0, 0)),
            pl.BlockSpec(memory_space=pltpu.SMEM),
        ],
        out_specs=pl.BlockSpec((TC_BLK, 8), lambda i: (i, 0)),
        out_shape=jax.ShapeDtypeStruct((ROWS128, 8), jnp.float32),
    )(table128, m2, bias)


_sc_mesh = plsc.VectorSubcoreMesh(core_axis_name="c", subcore_axis_name="s")


@functools.partial(
    pl.kernel,
    out_type=jax.ShapeDtypeStruct((B,), jnp.float32),
    mesh=_sc_mesh,
    scratch_types=[
        pltpu.VMEM((S, CH), jnp.int32),
        pltpu.VMEM((S, CH), jnp.float32),
        pltpu.VMEM((CH,), jnp.float32),
        pltpu.VMEM_SHARED((V,), jnp.float32),
        pltpu.SemaphoreType.DMA,
    ],
)
def _sc_pool(idx_hbm, t2_hbm, out_hbm, idx_v, vals_v, out_v, t2_sh, sem):
    cid = lax.axis_index("c")
    sid = lax.axis_index("s")
    wid = sid * NC + cid

    pltpu.sync_copy(t2_hbm.at[pl.ds(sid * STAGE, STAGE)],
                    t2_sh.at[pl.ds(sid * STAGE, STAGE)])

    @pl.when(sid == NS - 1)
    def _():
        pltpu.sync_copy(t2_hbm.at[pl.ds(STAGE_TAIL_OFF, STAGE_TAIL)],
                        t2_sh.at[pl.ds(STAGE_TAIL_OFF, STAGE_TAIL)])

    plsc.subcore_barrier()

    for c in range(NCHUNK):
        pltpu.sync_copy(idx_hbm.at[wid, c], idx_v)
        pltpu.async_copy(t2_sh.at[idx_v], vals_v, sem).wait()

        def body(s, accs):
            return tuple(accs[j] + vals_v[s, pl.ds(j * 16, 16)]
                         for j in range(CH // 16))

        accs = lax.fori_loop(
            0, S, body,
            tuple(jnp.zeros((16,), jnp.float32) for _ in range(CH // 16)))
        for j in range(CH // 16):
            out_v[pl.ds(j * 16, 16)] = 1.0 / (1.0 + jnp.exp(-accs[j]))
        pltpu.sync_copy(out_v, out_hbm.at[pl.ds(wid * ROWS_W + c * CH, CH)])


def kernel(inputs, table, dense_w, dense_b):
    w = dense_w[:, 0]
    m2 = (jnp.repeat(jnp.eye(8, dtype=jnp.float32), E, axis=0)
          * jnp.tile(w, 8)[:, None]) * (1.0 / S)
    bias = (dense_b * (1.0 / S)).reshape(1, 1)

    t2 = _fold_table(table.reshape(ROWS128, 128), m2, bias).reshape(V)

    idx_r = inputs.reshape(NW, NCHUNK, CH, S).swapaxes(2, 3)

    out = _sc_pool(idx_r, t2)
    return out.reshape(B, 1)


# speedup vs baseline: 9.3981x; 9.3981x over previous
import functools

import jax
import jax.numpy as jnp
from jax import lax
from jax.experimental import pallas as pl
from jax.experimental.pallas import tpu as pltpu
from jax.experimental.pallas import tpu_sc as plsc

V = 1000000
E = 16
B = 16384
S = 200

NC = 2
NS = 16
NW = NC * NS
ROWS_W = B // NW
CH = 128
NCHUNK = ROWS_W // CH

STAGE = 62496
STAGE_SUB = 20832
STAGE_TAIL_OFF = NS * STAGE
STAGE_TAIL = V - STAGE_TAIL_OFF


ROWS128 = V * E // 128
TC_BLK = 5000


def _tc_body(tab_ref, m2_ref, bias_ref, out_ref):
    out_ref[...] = (
        lax.dot_general(tab_ref[...], m2_ref[...], (((1,), (0,)), ((), ())),
                        preferred_element_type=jnp.float32)
        + bias_ref[0, 0]
    )


def _fold_table(table128, m2, bias):
    return pl.pallas_call(
        _tc_body,
        grid=(ROWS128 // TC_BLK,),
        in_specs=[
            pl.BlockSpec((TC_BLK, 128), lambda i: (i, 0)),
            pl.BlockSpec((128, 8), lambda i: (0, 0)),
            pl.BlockSpec(memory_space=pltpu.SMEM),
        ],
        out_specs=pl.BlockSpec((TC_BLK, 8), lambda i: (i, 0)),
        out_shape=jax.ShapeDtypeStruct((ROWS128, 8), jnp.float32),
    )(table128, m2, bias)


_sc_mesh = plsc.VectorSubcoreMesh(core_axis_name="c", subcore_axis_name="s")


@functools.partial(
    pl.kernel,
    out_type=jax.ShapeDtypeStruct((B,), jnp.float32),
    mesh=_sc_mesh,
    scratch_types=[
        pltpu.VMEM((S * CH,), jnp.int32),
        pltpu.VMEM((S * CH,), jnp.float32),
        pltpu.VMEM((CH,), jnp.float32),
        pltpu.VMEM_SHARED((V,), jnp.float32),
        pltpu.SemaphoreType.DMA,
    ],
)
def _sc_pool(idx_hbm, t2_hbm, out_hbm, idx_v, vals_v, out_v, t2_sh, sem):
    cid = lax.axis_index("c")
    sid = lax.axis_index("s")
    wid = sid * NC + cid

    for k in range(STAGE // STAGE_SUB):
        off = sid * STAGE + k * STAGE_SUB
        pltpu.sync_copy(t2_hbm.at[pl.ds(off, STAGE_SUB)],
                        vals_v.at[pl.ds(0, STAGE_SUB)])
        pltpu.sync_copy(vals_v.at[pl.ds(0, STAGE_SUB)],
                        t2_sh.at[pl.ds(off, STAGE_SUB)])

    @pl.when(sid == NS - 1)
    def _():
        pltpu.sync_copy(t2_hbm.at[pl.ds(STAGE_TAIL_OFF, STAGE_TAIL)],
                        vals_v.at[pl.ds(0, STAGE_TAIL)])
        pltpu.sync_copy(vals_v.at[pl.ds(0, STAGE_TAIL)],
                        t2_sh.at[pl.ds(STAGE_TAIL_OFF, STAGE_TAIL)])

    plsc.subcore_barrier()

    for c in range(NCHUNK):
        pltpu.sync_copy(idx_hbm.at[wid, c], idx_v)
        pltpu.async_copy(t2_sh.at[idx_v], vals_v, sem).wait()

        def body(s, accs):
            return tuple(accs[j] + vals_v[pl.ds(s * CH + j * 16, 16)]
                         for j in range(CH // 16))

        accs = lax.fori_loop(
            0, S, body,
            tuple(jnp.zeros((16,), jnp.float32) for _ in range(CH // 16)))
        for j in range(CH // 16):
            out_v[pl.ds(j * 16, 16)] = 1.0 / (1.0 + jnp.exp(-accs[j]))
        pltpu.sync_copy(out_v, out_hbm.at[pl.ds(wid * ROWS_W + c * CH, CH)])


def kernel(inputs, table, dense_w, dense_b):
    w = dense_w[:, 0]
    m2 = (jnp.repeat(jnp.eye(8, dtype=jnp.float32), E, axis=0)
          * jnp.tile(w, 8)[:, None]) * (1.0 / S)
    bias = (dense_b * (1.0 / S)).reshape(1, 1)

    t2 = _fold_table(table.reshape(ROWS128, 128), m2, bias).reshape(V)

    idx_r = inputs.reshape(NW, NCHUNK, CH, S).swapaxes(2, 3).reshape(
        NW, NCHUNK, S * CH)

    out = _sc_pool(idx_r, t2)
    return out.reshape(B, 1)


# speedup vs baseline: 9.4896x; 1.0097x over previous
import functools

import jax
import jax.numpy as jnp
from jax import lax
from jax.experimental import pallas as pl
from jax.experimental.pallas import tpu as pltpu
from jax.experimental.pallas import tpu_sc as plsc

V = 1000000
E = 16
B = 16384
S = 200

NC = 2
NS = 16
NW = NC * NS
ROWS_W = B // NW
CH = 128
NCHUNK = ROWS_W // CH

STAGE = 62496
STAGE_SUB = 20832
STAGE_TAIL_OFF = NS * STAGE
STAGE_TAIL = V - STAGE_TAIL_OFF


ROWS128 = V * E // 128
TC_BLK = 5000


def _tc_body(tab_ref, m2_ref, bias_ref, out_ref):
    out_ref[...] = (
        lax.dot_general(tab_ref[...], m2_ref[...], (((1,), (0,)), ((), ())),
                        preferred_element_type=jnp.float32)
        + bias_ref[0, 0]
    )


def _fold_table(table128, m2, bias):
    return pl.pallas_call(
        _tc_body,
        grid=(ROWS128 // TC_BLK,),
        in_specs=[
            pl.BlockSpec((TC_BLK, 128), lambda i: (i, 0)),
            pl.BlockSpec((128, 8), lambda i: (0, 0)),
            pl.BlockSpec(memory_space=pltpu.SMEM),
        ],
        out_specs=pl.BlockSpec((TC_BLK, 8), lambda i: (i, 0)),
        out_shape=jax.ShapeDtypeStruct((ROWS128, 8), jnp.float32),
    )(table128, m2, bias)


_sc_mesh = plsc.VectorSubcoreMesh(core_axis_name="c", subcore_axis_name="s")


@functools.partial(
    pl.kernel,
    out_type=jax.ShapeDtypeStruct((B,), jnp.float32),
    mesh=_sc_mesh,
    scratch_types=[
        pltpu.VMEM((S * CH,), jnp.int32),
        pltpu.VMEM((S * CH,), jnp.float32),
        pltpu.VMEM((CH,), jnp.float32),
        pltpu.VMEM_SHARED((V,), jnp.float32),
        pltpu.SemaphoreType.DMA,
    ],
    compiler_params=pltpu.CompilerParams(needs_layout_passes=False),
)
def _sc_pool(idx_hbm, t2_hbm, out_hbm, idx_v, vals_v, out_v, t2_sh, sem):
    cid = lax.axis_index("c")
    sid = lax.axis_index("s")
    wid = sid * NC + cid

    for k in range(STAGE // STAGE_SUB):
        off = sid * STAGE + k * STAGE_SUB
        pltpu.sync_copy(t2_hbm.at[pl.ds(off, STAGE_SUB)],
                        vals_v.at[pl.ds(0, STAGE_SUB)])
        pltpu.sync_copy(vals_v.at[pl.ds(0, STAGE_SUB)],
                        t2_sh.at[pl.ds(off, STAGE_SUB)])

    @pl.when(sid == NS - 1)
    def _():
        pltpu.sync_copy(t2_hbm.at[pl.ds(STAGE_TAIL_OFF, STAGE_TAIL)],
                        vals_v.at[pl.ds(0, STAGE_TAIL)])
        pltpu.sync_copy(vals_v.at[pl.ds(0, STAGE_TAIL)],
                        t2_sh.at[pl.ds(STAGE_TAIL_OFF, STAGE_TAIL)])

    plsc.subcore_barrier()

    lane = lax.iota(jnp.int32, 16)
    rowbase = tuple((j * 16 + lane) * S for j in range(CH // 16))

    for c in range(NCHUNK):
        pltpu.sync_copy(idx_hbm.at[wid, c], idx_v)
        pltpu.async_copy(t2_sh.at[idx_v], vals_v, sem).wait()

        def body(s, accs):
            return tuple(accs[j] + plsc.load_gather(vals_v, [rowbase[j] + s])
                         for j in range(CH // 16))

        accs = lax.fori_loop(
            0, S, body,
            tuple(jnp.zeros((16,), jnp.float32) for _ in range(CH // 16)))
        for j in range(CH // 16):
            out_v[pl.ds(j * 16, 16)] = 1.0 / (1.0 + jnp.exp(-accs[j]))
        pltpu.sync_copy(out_v, out_hbm.at[pl.ds(wid * ROWS_W + c * CH, CH)])


def kernel(inputs, table, dense_w, dense_b):
    w = dense_w[:, 0]
    m2 = (jnp.repeat(jnp.eye(8, dtype=jnp.float32), E, axis=0)
          * jnp.tile(w, 8)[:, None]) * (1.0 / S)
    bias = (dense_b * (1.0 / S)).reshape(1, 1)

    t2 = _fold_table(table.reshape(ROWS128, 128), m2, bias).reshape(V)

    idx_r = inputs.reshape(NW, NCHUNK, CH * S)

    out = _sc_pool(idx_r, t2)
    return out.reshape(B, 1)


# speedup vs baseline: 9.9258x; 1.0460x over previous
import functools

import jax
import jax.numpy as jnp
from jax import lax
from jax.experimental import pallas as pl
from jax.experimental.pallas import tpu as pltpu
from jax.experimental.pallas import tpu_sc as plsc

V = 1000000
E = 16
B = 16384
S = 200

NC = 2
NS = 16
NW = NC * NS
ROWS_W = B // NW
CH = 64
NCHUNK = ROWS_W // CH
CSZ = CH * S

STAGE_COLS = 7808
STAGE_CHUNKS = tuple((k * 1024, 1024) for k in range(7)) + ((7168, 640),)
STAGE_SUB_MAX = 1024
STAGE_TAIL_C = 124928


ROWS128 = V * E // 128
TC_BLK = 4096

T2C = ROWS128
T2C_PAD = 125056


def _tc_body(tab_ref, m2_ref, bias_ref, out_ref):
    out_ref[...] = (
        lax.dot_general(m2_ref[...], tab_ref[...], (((0,), (1,)), ((), ())),
                        preferred_element_type=jnp.float32)
        + bias_ref[0, 0]
    )


def _fold_table(table128, m2, bias):
    return pl.pallas_call(
        _tc_body,
        grid=((ROWS128 + TC_BLK - 1) // TC_BLK,),
        in_specs=[
            pl.BlockSpec((TC_BLK, 128), lambda i: (i, 0)),
            pl.BlockSpec((128, 8), lambda i: (0, 0)),
            pl.BlockSpec(memory_space=pltpu.SMEM),
        ],
        out_specs=pl.BlockSpec((8, TC_BLK), lambda i: (0, i)),
        out_shape=jax.ShapeDtypeStruct((8, T2C_PAD), jnp.float32),
    )(table128, m2, bias)


_sc_mesh = plsc.VectorSubcoreMesh(core_axis_name="c", subcore_axis_name="s")


@functools.partial(
    pl.kernel,
    out_type=jax.ShapeDtypeStruct((B,), jnp.float32),
    mesh=_sc_mesh,
    scratch_types=[
        pltpu.VMEM((CH, S), jnp.int32),
        pltpu.VMEM((CSZ,), jnp.int32),
        pltpu.VMEM((CSZ,), jnp.float32),
        pltpu.VMEM((8, STAGE_SUB_MAX), jnp.float32),
        pltpu.VMEM((8 * STAGE_SUB_MAX,), jnp.float32),
        pltpu.VMEM((CH,), jnp.float32),
        pltpu.VMEM_SHARED((8 * T2C_PAD,), jnp.float32),
        pltpu.SemaphoreType.DMA,
    ],
    compiler_params=pltpu.CompilerParams(needs_layout_passes=False),
)
def _sc_pool(idx_hbm, t2_hbm, out_hbm, idx_t, idx_v, vals_v, t2t_v, stage_v,
             out_v, t2_sh, sem):
    cid = lax.axis_index("c")
    sid = lax.axis_index("s")
    wid = sid * NC + cid

    lane = lax.iota(jnp.int32, 16)
    lane8 = lane * 8

    def _stage(col0, cols):
        pltpu.sync_copy(t2_hbm.at[:, pl.ds(col0, cols)],
                        t2t_v.at[:, pl.ds(0, cols)])

        def depile(t, carry):
            i0 = jnp.maximum(jnp.minimum(t * 16, cols - 16), 0)
            for g in range(8):
                x = t2t_v[g, pl.ds(i0, 16)]
                plsc.store_scatter(stage_v, [i0 * 8 + lane8 + g], x)
            return carry

        lax.fori_loop(0, (cols + 15) // 16, depile, 0)
        pltpu.sync_copy(stage_v.at[pl.ds(0, cols * 8)],
                        t2_sh.at[pl.ds(col0 * 8, cols * 8)])

    for c0off, cl in STAGE_CHUNKS:
        _stage(sid * STAGE_COLS + c0off, cl)

    @pl.when(sid == NS - 1)
    def _():
        _stage(STAGE_TAIL_C, 128)

    plsc.subcore_barrier()

    lane = lax.iota(jnp.int32, 16)
    rowbase = tuple((j * 16 + lane) * S for j in range(CH // 16))

    for c in range(NCHUNK):
        row0 = (wid * NCHUNK + c) * CH
        pltpu.sync_copy(idx_hbm.at[pl.ds(row0, CH), :], idx_t)

        def repack(r, carry):
            for k in range(S // 16):
                idx_v[pl.ds(r * S + k * 16, 16)] = idx_t[r, pl.ds(k * 16, 16)]
            idx_v[pl.ds(r * S + S - 16, 16)] = idx_t[r, pl.ds(S - 16, 16)]
            return carry

        lax.fori_loop(0, CH, repack, 0)

        pltpu.async_copy(t2_sh.at[idx_v], vals_v, sem).wait()

        def body(s, accs):
            return tuple(accs[j] + plsc.load_gather(vals_v, [rowbase[j] + s])
                         for j in range(CH // 16))

        accs = lax.fori_loop(
            0, S, body,
            tuple(jnp.zeros((16,), jnp.float32) for _ in range(CH // 16)))
        for j in range(CH // 16):
            out_v[pl.ds(j * 16, 16)] = 1.0 / (1.0 + jnp.exp(-accs[j]))
        pltpu.sync_copy(out_v, out_hbm.at[pl.ds(wid * ROWS_W + c * CH, CH)])


def kernel(inputs, table, dense_w, dense_b):
    w = dense_w[:, 0]
    m2 = (jnp.repeat(jnp.eye(8, dtype=jnp.float32), E, axis=0)
          * jnp.tile(w, 8)[:, None]) * (1.0 / S)
    bias = (dense_b * (1.0 / S)).reshape(1, 1)

    t2 = _fold_table(table.reshape(ROWS128, 128), m2, bias)

    out = _sc_pool(inputs, t2)
    return out.reshape(B, 1)


# speedup vs baseline: 17.4362x; 1.7567x over previous
import functools

import jax
import jax.numpy as jnp
from jax import lax
from jax.experimental import pallas as pl
from jax.experimental.pallas import tpu as pltpu
from jax.experimental.pallas import tpu_sc as plsc

V = 1000000
E = 16
B = 16384
S = 200

NC = 2
NS = 16
NW = NC * NS
ROWS_W = B // NW
CH = 128
NCHUNK = ROWS_W // CH
SH = S // 2
CSZH = CH * SH

VP = 1001472
STAGE = VP // NS
STAGE_SUB = STAGE // 8


TC_BLK = 2048


def _tc_body(tab_ref, w_ref, bias_ref, out_ref):
    out_ref[...] = (
        jnp.sum(tab_ref[...] * w_ref[...], axis=0) + bias_ref[0, 0]
    )


def _fold_table(table_t, w_col, bias):
    return pl.pallas_call(
        _tc_body,
        grid=(VP // TC_BLK,),
        in_specs=[
            pl.BlockSpec((E, TC_BLK), lambda k: (0, k)),
            pl.BlockSpec((E, 1), lambda k: (0, 0)),
            pl.BlockSpec(memory_space=pltpu.SMEM),
        ],
        out_specs=pl.BlockSpec((TC_BLK,), lambda k: (k,)),
        out_shape=jax.ShapeDtypeStruct((VP,), jnp.float32),
    )(table_t, w_col, bias)


_sc_mesh = plsc.VectorSubcoreMesh(core_axis_name="c", subcore_axis_name="s")


@functools.partial(
    pl.kernel,
    out_type=jax.ShapeDtypeStruct((B,), jnp.float32),
    mesh=_sc_mesh,
    scratch_types=[
        pltpu.VMEM((S, CH), jnp.int32),
        pltpu.VMEM((CSZH,), jnp.int32),
        pltpu.VMEM((CSZH,), jnp.float32),
        pltpu.VMEM((STAGE_SUB,), jnp.float32),
        pltpu.VMEM((CH,), jnp.float32),
        pltpu.VMEM_SHARED((VP,), jnp.float32),
        pltpu.SemaphoreType.DMA,
    ],
    compiler_params=pltpu.CompilerParams(needs_layout_passes=False),
)
def _sc_pool(idx_hbm, t2_hbm, out_hbm, idx_t, idx_v, vals_v, stage_v, out_v,
             t2_sh, sem):
    cid = lax.axis_index("c")
    sid = lax.axis_index("s")
    wid = sid * NC + cid

    for k in range(STAGE // STAGE_SUB):
        off = sid * STAGE + k * STAGE_SUB
        pltpu.sync_copy(t2_hbm.at[pl.ds(off, STAGE_SUB)], stage_v)
        pltpu.sync_copy(stage_v, t2_sh.at[pl.ds(off, STAGE_SUB)])

    plsc.subcore_barrier()

    NG = CH // 16
    for c in range(NCHUNK):
        col0 = wid * ROWS_W + c * CH
        pltpu.sync_copy(idx_hbm.at[:, pl.ds(col0, CH)], idx_t)

        accs = [jnp.zeros((16,), jnp.float32)] * NG
        for h in range(S // SH):
            def repack(s, carry):
                for g in range(NG):
                    idx_v[pl.ds(s * CH + g * 16, 16)] = (
                        idx_t[h * SH + s, pl.ds(g * 16, 16)])
                return carry

            lax.fori_loop(0, SH, repack, 0)
            pltpu.async_copy(t2_sh.at[idx_v], vals_v, sem).wait()

            def body(s, a):
                return tuple(a[g] + vals_v[pl.ds(s * CH + g * 16, 16)]
                             for g in range(NG))

            accs = list(lax.fori_loop(0, SH, body, tuple(accs)))

        for g in range(NG):
            out_v[pl.ds(g * 16, 16)] = 1.0 / (1.0 + jnp.exp(-accs[g]))
        pltpu.sync_copy(out_v, out_hbm.at[pl.ds(col0, CH)])


def kernel(inputs, table, dense_w, dense_b):
    w_col = dense_w * (1.0 / S)
    bias = (dense_b * (1.0 / S)).reshape(1, 1)

    t2 = _fold_table(table.T, w_col, bias)

    out = _sc_pool(inputs.T, t2)
    return out.reshape(B, 1)


# speedup vs baseline: 52.3677x; 3.0034x over previous
import functools

import jax
import jax.numpy as jnp
from jax import lax
from jax.experimental import pallas as pl
from jax.experimental.pallas import tpu as pltpu
from jax.experimental.pallas import tpu_sc as plsc

V = 1000000
E = 16
B = 16384
S = 200

NC = 2
NS = 16
NW = NC * NS
ROWS_W = B // NW
CH = 128
NCHUNK = ROWS_W // CH
SH = S // 2
CSZH = CH * SH

VP = 1015808
STAGE = VP // NS
STAGE_SUB = STAGE // 8


TC_BLK = 32768


def _tc_body(tab_ref, w_ref, bias_ref, out_ref):
    out_ref[...] = (
        jnp.sum(tab_ref[...] * w_ref[...], axis=0) + bias_ref[0, 0]
    )


def _fold_table(table_t, w_col, bias):
    return pl.pallas_call(
        _tc_body,
        grid=(VP // TC_BLK,),
        in_specs=[
            pl.BlockSpec((E, TC_BLK), lambda k: (0, k)),
            pl.BlockSpec((E, 1), lambda k: (0, 0)),
            pl.BlockSpec(memory_space=pltpu.SMEM),
        ],
        out_specs=pl.BlockSpec((TC_BLK,), lambda k: (k,)),
        out_shape=jax.ShapeDtypeStruct((VP,), jnp.float32),
    )(table_t, w_col, bias)


_sc_mesh = plsc.VectorSubcoreMesh(core_axis_name="c", subcore_axis_name="s")


@functools.partial(
    pl.kernel,
    out_type=jax.ShapeDtypeStruct((B,), jnp.float32),
    mesh=_sc_mesh,
    scratch_types=[
        pltpu.VMEM((S, CH), jnp.int32),
        pltpu.VMEM((CSZH,), jnp.int32),
        pltpu.VMEM((CSZH,), jnp.float32),
        pltpu.VMEM((STAGE_SUB,), jnp.float32),
        pltpu.VMEM((CH,), jnp.float32),
        pltpu.VMEM_SHARED((VP,), jnp.float32),
        pltpu.SemaphoreType.DMA,
    ],
    compiler_params=pltpu.CompilerParams(needs_layout_passes=False),
)
def _sc_pool(idx_hbm, t2_hbm, out_hbm, idx_t, idx_v, vals_v, stage_v, out_v,
             t2_sh, sem):
    cid = lax.axis_index("c")
    sid = lax.axis_index("s")
    wid = sid * NC + cid

    for k in range(STAGE // STAGE_SUB):
        off = sid * STAGE + k * STAGE_SUB
        pltpu.sync_copy(t2_hbm.at[pl.ds(off, STAGE_SUB)], stage_v)
        pltpu.sync_copy(stage_v, t2_sh.at[pl.ds(off, STAGE_SUB)])

    plsc.subcore_barrier()

    NG = CH // 16
    for c in range(NCHUNK):
        col0 = wid * ROWS_W + c * CH
        pltpu.sync_copy(idx_hbm.at[:, pl.ds(col0, CH)], idx_t)

        accs = [jnp.zeros((16,), jnp.float32)] * NG
        for h in range(S // SH):
            def repack(s, carry):
                for g in range(NG):
                    idx_v[pl.ds(s * CH + g * 16, 16)] = (
                        idx_t[h * SH + s, pl.ds(g * 16, 16)])
                return carry

            lax.fori_loop(0, SH, repack, 0)
            pltpu.async_copy(t2_sh.at[idx_v], vals_v, sem).wait()

            def body(s, a):
                return tuple(a[g] + vals_v[pl.ds(s * CH + g * 16, 16)]
                             for g in range(NG))

            accs = list(lax.fori_loop(0, SH, body, tuple(accs)))

        for g in range(NG):
            out_v[pl.ds(g * 16, 16)] = 1.0 / (1.0 + jnp.exp(-accs[g]))
        pltpu.sync_copy(out_v, out_hbm.at[pl.ds(col0, CH)])


def kernel(inputs, table, dense_w, dense_b):
    w_col = dense_w * (1.0 / S)
    bias = (dense_b * (1.0 / S)).reshape(1, 1)

    t2 = _fold_table(table.T, w_col, bias)

    out = _sc_pool(inputs.T, t2)
    return out.reshape(B, 1)


# speedup vs baseline: 63.4554x; 1.2117x over previous
import functools

import jax
import jax.numpy as jnp
from jax import lax
from jax.experimental import pallas as pl
from jax.experimental.pallas import tpu as pltpu
from jax.experimental.pallas import tpu_sc as plsc

V = 1000000
E = 16
B = 16384
S = 200

NC = 2
NS = 16
NW = NC * NS
ROWS_W = B // NW
CH = 128
NCHUNK = ROWS_W // CH
NPASS = 4
SH = S // NPASS
CSZH = CH * SH

VP = 1048576
STAGE = VP // NS
STAGE_SUB = STAGE // 8


TC_BLK = 65536


def _tc_body(tab_ref, w_ref, bias_ref, out_ref):
    out_ref[...] = (
        jnp.sum(tab_ref[...] * w_ref[...], axis=0) + bias_ref[0, 0]
    )


def _fold_table(table_t, w_col, bias):
    return pl.pallas_call(
        _tc_body,
        grid=(VP // TC_BLK,),
        in_specs=[
            pl.BlockSpec((E, TC_BLK), lambda k: (0, k)),
            pl.BlockSpec((E, 1), lambda k: (0, 0)),
            pl.BlockSpec(memory_space=pltpu.SMEM),
        ],
        out_specs=pl.BlockSpec((TC_BLK,), lambda k: (k,)),
        out_shape=jax.ShapeDtypeStruct((VP,), jnp.float32),
    )(table_t, w_col, bias)


_sc_mesh = plsc.VectorSubcoreMesh(core_axis_name="c", subcore_axis_name="s")


@functools.partial(
    pl.kernel,
    out_type=jax.ShapeDtypeStruct((B,), jnp.float32),
    mesh=_sc_mesh,
    scratch_types=[
        pltpu.VMEM((S, CH), jnp.int32),
        pltpu.VMEM((CSZH,), jnp.int32),
        pltpu.VMEM((CSZH,), jnp.int32),
        pltpu.VMEM((CSZH,), jnp.float32),
        pltpu.VMEM((CSZH,), jnp.float32),
        pltpu.VMEM((STAGE_SUB,), jnp.float32),
        pltpu.VMEM((CH,), jnp.float32),
        pltpu.VMEM_SHARED((VP,), jnp.float32),
        pltpu.SemaphoreType.DMA,
        pltpu.SemaphoreType.DMA,
    ],
    compiler_params=pltpu.CompilerParams(needs_layout_passes=False),
)
def _sc_pool(idx_hbm, t2_hbm, out_hbm, idx_t, idx_v0, idx_v1, vals_v0,
             vals_v1, stage_v, out_v, t2_sh, sem, sem2):
    cid = lax.axis_index("c")
    sid = lax.axis_index("s")
    wid = sid * NC + cid
    idx_vs = (idx_v0, idx_v1)
    vals_vs = (vals_v0, vals_v1)

    def fire_idx(c):
        col0 = wid * ROWS_W + c * CH
        return pltpu.async_copy(idx_hbm.at[:, pl.ds(col0, CH)], idx_t, sem2)

    idx_dma = fire_idx(0)

    for k in range(STAGE // STAGE_SUB):
        off = sid * STAGE + k * STAGE_SUB
        pltpu.sync_copy(t2_hbm.at[pl.ds(off, STAGE_SUB)], stage_v)
        pltpu.sync_copy(stage_v, t2_sh.at[pl.ds(off, STAGE_SUB)])

    plsc.subcore_barrier()

    NG = CH // 16
    for c in range(NCHUNK):
        col0 = wid * ROWS_W + c * CH
        idx_dma.wait()

        accs = [jnp.zeros((16,), jnp.float32)] * NG

        def reduce_pass(b, accs):
            vv = vals_vs[b]

            def body(s, a):
                return tuple(a[g] + vv[pl.ds(s * CH + g * 16, 16)]
                             for g in range(NG))

            return list(lax.fori_loop(0, SH, body, tuple(accs)))

        gat = None
        for p in range(NPASS):
            b = p & 1
            iv = idx_vs[b]

            def repack(s, carry, p=p, iv=iv):
                for g in range(NG):
                    iv[pl.ds(s * CH + g * 16, 16)] = (
                        idx_t[p * SH + s, pl.ds(g * 16, 16)])
                return carry

            lax.fori_loop(0, SH, repack, 0)
            d = pltpu.async_copy(t2_sh.at[iv], vals_vs[b], sem)
            if p == NPASS - 1 and c < NCHUNK - 1:
                idx_dma = fire_idx(c + 1)
            if gat is not None:
                gat.wait()
                accs = reduce_pass(1 - b, accs)
            gat = d
        gat.wait()
        accs = reduce_pass((NPASS - 1) & 1, accs)

        for g in range(NG):
            out_v[pl.ds(g * 16, 16)] = 1.0 / (1.0 + jnp.exp(-accs[g]))
        pltpu.sync_copy(out_v, out_hbm.at[pl.ds(col0, CH)])


def kernel(inputs, table, dense_w, dense_b):
    w_col = dense_w * (1.0 / S)
    bias = (dense_b * (1.0 / S)).reshape(1, 1)

    t2 = _fold_table(table.T, w_col, bias)

    out = _sc_pool(inputs.T, t2)
    return out.reshape(B, 1)
